# baseline (speedup 1.0000x reference)
import functools
import math

import jax
import jax.numpy as jnp
from jax import lax
from jax.experimental import pallas as pl
from jax.experimental.pallas import tpu as pltpu
from jax.experimental.pallas import tpu_sc as plsc

D_MODEL = 64
SCALE = math.sqrt(D_MODEL)
CHUNK = 128


def _emb_kernel(idx_hbm, tab_hbm, out_hbm, idx_v, rows_v, sem, *, n_chunks,
                per_w, nc):
    wid = lax.axis_index("s") * nc + lax.axis_index("c")
    base = wid * per_w
    pltpu.sync_copy(idx_hbm.at[wid], idx_v)

    def chunk_body(c, carry):
        pltpu.async_copy(tab_hbm.at[idx_v.at[c]], rows_v, sem).wait()

        def scale_row(r, carry2):
            for q in range(D_MODEL // 16):
                rows_v[r, pl.ds(q * 16, 16)] = (
                    rows_v[r, pl.ds(q * 16, 16)] * SCALE)
            return carry2

        lax.fori_loop(0, CHUNK, scale_row, 0, unroll=2)
        pltpu.sync_copy(rows_v, out_hbm.at[pl.ds(base + c * CHUNK, CHUNK)])
        return carry

    lax.fori_loop(0, n_chunks, chunk_body, 0)


def kernel(inputs, table):
    n_tok, seq = inputs.shape
    b_total = n_tok * seq
    info = plsc.get_sparse_core_info()
    nc, ns = info.num_cores, info.num_subcores
    nw = nc * ns
    per_w = b_total // nw
    n_chunks = per_w // CHUNK
    assert per_w * nw == b_total and n_chunks * CHUNK == per_w

    idx = inputs.astype(jnp.int32).reshape(nw, n_chunks, CHUNK)

    mesh = plsc.VectorSubcoreMesh(core_axis_name="c", subcore_axis_name="s")
    k = functools.partial(
        pl.kernel,
        out_type=jax.ShapeDtypeStruct((b_total, D_MODEL), jnp.float32),
        mesh=mesh,
        scratch_types=[
            pltpu.VMEM((n_chunks, CHUNK), jnp.int32),
            pltpu.VMEM((CHUNK, D_MODEL), jnp.float32),
            pltpu.SemaphoreType.DMA,
        ],
        compiler_params=pltpu.CompilerParams(use_tc_tiling_on_sc=False),
    )(functools.partial(_emb_kernel, n_chunks=n_chunks, per_w=per_w, nc=nc))

    out = k(idx, table)
    return out.reshape(n_tok, seq, D_MODEL)
